# baseline (speedup 1.0000x reference)
import functools

import jax
import jax.numpy as jnp
from jax import lax
from jax.experimental import pallas as pl
from jax.experimental.pallas import tpu as pltpu
from jax.experimental.pallas import tpu_sc as plsc

_LANES = 16
_NBUF = 3


def _sc_sum(word_indices, packed_table):
    B, S = word_indices.shape
    EP = packed_table.shape[1]
    E = 2 * EP
    NC, NS = 2, 16
    NW = NC * NS
    R = B // NW
    GV = EP // _LANES
    C0 = 104
    C1 = S - C0
    assert C0 % 8 == 0 and C1 <= 128 and S % 8 == 0
    NB = _NBUF
    MAIN = (R // NB) * NB

    mesh = plsc.VectorSubcoreMesh(core_axis_name="c", subcore_axis_name="s")

    @functools.partial(
        pl.kernel,
        out_type=jax.ShapeDtypeStruct((B, E), jnp.float32),
        mesh=mesh,
        compiler_params=pltpu.CompilerParams(
            use_tc_tiling_on_sc=False, needs_layout_passes=False),
        scratch_types=[
            pltpu.VMEM((R, S), jnp.int32),
            pltpu.VMEM((NB, S, EP), jnp.int32),
            pltpu.VMEM((R, E), jnp.float32),
        ] + [pltpu.SemaphoreType.DMA] * NB,
    )
    def k(idx_hbm, table_hbm, out_hbm, idx_v, rows_v, sums_v, *sems):
        wid = lax.axis_index("s") * NC + lax.axis_index("c")
        base = wid * R

        pltpu.sync_copy(idx_hbm.at[pl.ds(base, R)], idx_v)

        def gather_row(row, buf):
            sem = sems[buf]
            pltpu.async_copy(
                table_hbm.at[idx_v.at[row, pl.ds(0, C0)]],
                rows_v.at[buf, pl.ds(0, C0), :], sem)
            pltpu.async_copy(
                table_hbm.at[idx_v.at[row, pl.ds(C0, C1)]],
                rows_v.at[buf, pl.ds(C0, C1), :], sem)

        def wait_row(buf):
            sem = sems[buf]
            pltpu.make_async_copy(
                table_hbm.at[idx_v.at[0, pl.ds(0, C0)]],
                rows_v.at[buf, pl.ds(0, C0), :], sem).wait()
            pltpu.make_async_copy(
                table_hbm.at[idx_v.at[0, pl.ds(C0, C1)]],
                rows_v.at[buf, pl.ds(C0, C1), :], sem).wait()

        def accumulate(row, buf):
            def body(t, acc):
                acc = list(acc)
                for j in range(8):
                    s = t * 8 + j
                    for g in range(GV):
                        w = rows_v[buf, s, pl.ds(g * _LANES, _LANES)]
                        lo, hi = plsc.unpack(
                            plsc.bitcast(w, jnp.bfloat16),
                            format=plsc.PackFormat.INTERLEAVED)
                        acc[2 * g] = acc[2 * g] + lo.astype(jnp.float32)
                        acc[2 * g + 1] = acc[2 * g + 1] + hi.astype(
                            jnp.float32)
                return tuple(acc)
            acc = lax.fori_loop(
                0, S // 8, body,
                tuple(jnp.zeros((_LANES,), jnp.float32)
                      for _ in range(2 * GV)))
            for e in range(2 * GV):
                sums_v[row, pl.ds(e * _LANES, _LANES)] = acc[e]

        for b in range(NB):
            gather_row(b, b)

        def outer(g, carry):
            for b in range(NB):
                row = g + b
                wait_row(b)
                accumulate(row, b)
                gather_row(jnp.minimum(row + NB, R - 1), b)
            return carry

        lax.fori_loop(0, MAIN // NB, lambda t, c: outer(t * NB, c), 0)

        for b in range(NB):
            row = MAIN + b
            wait_row(b)
            if row < R:
                accumulate(row, b)

        pltpu.sync_copy(sums_v, out_hbm.at[pl.ds(base, R)])

    return k(word_indices, packed_table)


def _tc_finish(sums, word_indices, emb0, W1, b1, W2, b2):
    B, S = word_indices.shape
    E = sums.shape[1]
    H = W1.shape[1]
    C = W2.shape[1]
    BB = B
    grid = B // BB

    def body(sums_ref, idx_ref, emb0_ref, w1_ref, b1_ref, w2_ref, b2_ref,
             out_ref):
        idx = idx_ref[...]
        cnt = jnp.sum((idx != 0).astype(jnp.float32), axis=1, keepdims=True)
        npad = float(S) - cnt
        summed = sums_ref[...] - npad * emb0_ref[...]
        avg = jnp.where(cnt > 0, summed / jnp.maximum(cnt, 1.0), 0.0)
        hidden = jnp.maximum(
            jnp.dot(avg, w1_ref[...], preferred_element_type=jnp.float32)
            + b1_ref[...], 0.0)
        out_ref[...] = (
            jnp.dot(hidden, w2_ref[...], preferred_element_type=jnp.float32)
            + b2_ref[...])

    return pl.pallas_call(
        body,
        grid=(grid,),
        in_specs=[
            pl.BlockSpec((BB, E), lambda i: (i, 0)),
            pl.BlockSpec((BB, S), lambda i: (i, 0)),
            pl.BlockSpec((1, E), lambda i: (0, 0)),
            pl.BlockSpec((E, H), lambda i: (0, 0)),
            pl.BlockSpec((1, H), lambda i: (0, 0)),
            pl.BlockSpec((H, C), lambda i: (0, 0)),
            pl.BlockSpec((1, C), lambda i: (0, 0)),
        ],
        out_specs=pl.BlockSpec((BB, C), lambda i: (i, 0)),
        out_shape=jax.ShapeDtypeStruct((B, C), jnp.float32),
    )(sums, word_indices, emb0, W1, b1, W2, b2)


def _perm(E):
    p = []
    for g in range(E // 32):
        p.extend(32 * g + 2 * k for k in range(16))
        p.extend(32 * g + 2 * k + 1 for k in range(16))
    return p


def kernel(word_indices, emb_table, W1, b1, W2, b2):
    idx = word_indices.astype(jnp.int32)
    E = emb_table.shape[1]
    table_bf = emb_table.astype(jnp.bfloat16)
    packed = lax.bitcast_convert_type(
        table_bf.reshape(emb_table.shape[0], E // 2, 2), jnp.int32)
    sums = _sc_sum(idx, packed)
    p = jnp.array(_perm(E), dtype=jnp.int32)
    emb0 = table_bf[0:1].astype(jnp.float32)[:, p]
    return _tc_finish(sums, idx, emb0, W1[p, :], b1.reshape(1, -1),
                      W2, b2.reshape(1, -1))


# speedup vs baseline: 2.4351x; 2.4351x over previous
import functools

import jax
import jax.numpy as jnp
from jax import lax
from jax.experimental import pallas as pl
from jax.experimental.pallas import tpu as pltpu
from jax.experimental.pallas import tpu_sc as plsc

_LANES = 16
_NBUF = 3


def _sc_sum(word_indices, table_bf):
    B, S = word_indices.shape
    E = table_bf.shape[1]
    NC, NS = 2, 16
    NW = NC * NS
    R = B // NW
    GV = E // (2 * _LANES)
    C0 = 104
    C1 = S - C0
    assert C0 % 8 == 0 and C1 <= 128 and S % 8 == 0
    NB = _NBUF
    MAIN = (R // NB) * NB

    mesh = plsc.VectorSubcoreMesh(core_axis_name="c", subcore_axis_name="s")

    @functools.partial(
        pl.kernel,
        out_type=jax.ShapeDtypeStruct((B, E), jnp.float32),
        mesh=mesh,
        compiler_params=pltpu.CompilerParams(
            use_tc_tiling_on_sc=False, needs_layout_passes=False),
        scratch_types=[
            pltpu.VMEM((R, S), jnp.int32),
            pltpu.VMEM((NB, S, E), jnp.bfloat16),
            pltpu.VMEM((R, E), jnp.float32),
        ] + [pltpu.SemaphoreType.DMA] * NB,
    )
    def k(idx_hbm, table_hbm, out_hbm, idx_v, rows_v, sums_v, *sems):
        wid = lax.axis_index("s") * NC + lax.axis_index("c")
        base = wid * R

        pltpu.sync_copy(idx_hbm.at[pl.ds(base, R)], idx_v)

        def gather_row(row, buf):
            sem = sems[buf]
            pltpu.async_copy(
                table_hbm.at[idx_v.at[row, pl.ds(0, C0)]],
                rows_v.at[buf, pl.ds(0, C0), :], sem)
            pltpu.async_copy(
                table_hbm.at[idx_v.at[row, pl.ds(C0, C1)]],
                rows_v.at[buf, pl.ds(C0, C1), :], sem)

        def wait_row(buf):
            sem = sems[buf]
            pltpu.make_async_copy(
                table_hbm.at[idx_v.at[0, pl.ds(0, C0)]],
                rows_v.at[buf, pl.ds(0, C0), :], sem).wait()
            pltpu.make_async_copy(
                table_hbm.at[idx_v.at[0, pl.ds(C0, C1)]],
                rows_v.at[buf, pl.ds(C0, C1), :], sem).wait()

        def accumulate(row, buf):
            def body(t, acc):
                acc = list(acc)
                for j in range(8):
                    s = t * 8 + j
                    for g in range(GV):
                        w = rows_v[buf, s, pl.ds(g * 2 * _LANES, 2 * _LANES)]
                        lo, hi = plsc.unpack(
                            w, format=plsc.PackFormat.INTERLEAVED)
                        acc[2 * g] = acc[2 * g] + lo.astype(jnp.float32)
                        acc[2 * g + 1] = acc[2 * g + 1] + hi.astype(
                            jnp.float32)
                return tuple(acc)
            acc = lax.fori_loop(
                0, S // 8, body,
                tuple(jnp.zeros((_LANES,), jnp.float32)
                      for _ in range(2 * GV)))
            for e in range(2 * GV):
                sums_v[row, pl.ds(e * _LANES, _LANES)] = acc[e]

        for b in range(NB):
            gather_row(b, b)

        def outer(g, carry):
            for b in range(NB):
                row = g + b
                wait_row(b)
                accumulate(row, b)
                gather_row(jnp.minimum(row + NB, R - 1), b)
            return carry

        lax.fori_loop(0, MAIN // NB, lambda t, c: outer(t * NB, c), 0)

        for b in range(NB):
            row = MAIN + b
            wait_row(b)
            if row < R:
                accumulate(row, b)

        pltpu.sync_copy(sums_v, out_hbm.at[pl.ds(base, R)])

    return k(word_indices, table_bf)


def _tc_finish(sums, word_indices, emb0, W1, b1, W2, b2):
    B, S = word_indices.shape
    E = sums.shape[1]
    H = W1.shape[1]
    C = W2.shape[1]
    BB = B
    grid = B // BB

    def body(sums_ref, idx_ref, emb0_ref, w1_ref, b1_ref, w2_ref, b2_ref,
             out_ref):
        idx = idx_ref[...]
        cnt = jnp.sum((idx != 0).astype(jnp.float32), axis=1, keepdims=True)
        npad = float(S) - cnt
        summed = sums_ref[...] - npad * emb0_ref[...]
        avg = jnp.where(cnt > 0, summed / jnp.maximum(cnt, 1.0), 0.0)
        hidden = jnp.maximum(
            jnp.dot(avg, w1_ref[...], preferred_element_type=jnp.float32)
            + b1_ref[...], 0.0)
        out_ref[...] = (
            jnp.dot(hidden, w2_ref[...], preferred_element_type=jnp.float32)
            + b2_ref[...])

    return pl.pallas_call(
        body,
        grid=(grid,),
        in_specs=[
            pl.BlockSpec((BB, E), lambda i: (i, 0)),
            pl.BlockSpec((BB, S), lambda i: (i, 0)),
            pl.BlockSpec((1, E), lambda i: (0, 0)),
            pl.BlockSpec((E, H), lambda i: (0, 0)),
            pl.BlockSpec((1, H), lambda i: (0, 0)),
            pl.BlockSpec((H, C), lambda i: (0, 0)),
            pl.BlockSpec((1, C), lambda i: (0, 0)),
        ],
        out_specs=pl.BlockSpec((BB, C), lambda i: (i, 0)),
        out_shape=jax.ShapeDtypeStruct((B, C), jnp.float32),
    )(sums, word_indices, emb0, W1, b1, W2, b2)


def _perm(E):
    p = []
    for g in range(E // 32):
        p.extend(32 * g + 2 * k for k in range(16))
        p.extend(32 * g + 2 * k + 1 for k in range(16))
    return p


def kernel(word_indices, emb_table, W1, b1, W2, b2):
    idx = word_indices.astype(jnp.int32)
    E = emb_table.shape[1]
    table_bf = emb_table.astype(jnp.bfloat16)
    sums = _sc_sum(idx, table_bf)
    p = jnp.array(_perm(E), dtype=jnp.int32)
    emb0 = table_bf[0:1].astype(jnp.float32)[:, p]
    return _tc_finish(sums, idx, emb0, W1[p, :], b1.reshape(1, -1),
                      W2, b2.reshape(1, -1))


# speedup vs baseline: 2.7600x; 1.1334x over previous
import functools

import jax
import jax.numpy as jnp
from jax import lax
from jax.experimental import pallas as pl
from jax.experimental.pallas import tpu as pltpu
from jax.experimental.pallas import tpu_sc as plsc

_LANES = 16
_NBUF = 3


def _sc_pack(emb_table):
    V, E = emb_table.shape
    NC, NS = 2, 16
    NW = NC * NS
    CW = 136
    NCH = 23
    RW = CW * NCH
    VC = V - CW
    assert VC % 8 == 0 and CW % 8 == 0 and RW % 8 == 0
    assert NW * RW >= V and (NW - 1) * RW <= VC
    GV = E // (2 * _LANES)

    mesh = plsc.VectorSubcoreMesh(core_axis_name="c", subcore_axis_name="s")

    @functools.partial(
        pl.kernel,
        out_type=jax.ShapeDtypeStruct((V, E), jnp.bfloat16),
        mesh=mesh,
        compiler_params=pltpu.CompilerParams(
            use_tc_tiling_on_sc=False, needs_layout_passes=False),
        scratch_types=[
            pltpu.VMEM((2, CW, E), jnp.float32),
            pltpu.VMEM((2, CW, E), jnp.bfloat16),
        ] + [pltpu.SemaphoreType.DMA] * 4,
    )
    def k(tab_hbm, out_hbm, in_v, out_v, in_s0, in_s1, out_s0, out_s1):
        wid = lax.axis_index("s") * NC + lax.axis_index("c")
        base = wid * RW
        in_sems = (in_s0, in_s1)
        out_sems = (out_s0, out_s1)

        def start(c):
            return jnp.minimum(base + c * CW, VC)

        def dma_in(c, buf):
            pltpu.async_copy(tab_hbm.at[pl.ds(start(c), CW)],
                             in_v.at[buf], in_sems[buf])

        def dma_out(c, buf):
            pltpu.async_copy(out_v.at[buf],
                             out_hbm.at[pl.ds(start(c), CW)], out_sems[buf])

        def wait_in(buf):
            pltpu.make_async_copy(tab_hbm.at[pl.ds(0, CW)],
                                  in_v.at[buf], in_sems[buf]).wait()

        def wait_out(buf):
            pltpu.make_async_copy(out_v.at[buf],
                                  out_hbm.at[pl.ds(0, CW)],
                                  out_sems[buf]).wait()

        def pack_chunk(buf):
            def row(r, carry):
                for g in range(GV):
                    a = in_v[buf, r, pl.ds(g * 2 * _LANES, _LANES)]
                    b = in_v[buf, r, pl.ds(g * 2 * _LANES + _LANES, _LANES)]
                    out_v[buf, r, pl.ds(g * 2 * _LANES, 2 * _LANES)] = (
                        plsc.pack(a, b, format=plsc.PackFormat.INTERLEAVED))
                return carry
            lax.fori_loop(0, CW, row, 0)

        def body(c, buf, first, prefetch):
            wait_in(buf)
            if not first:
                wait_out(buf)
            pack_chunk(buf)
            dma_out(c, buf)
            if prefetch:
                dma_in(c + 2, buf)

        dma_in(0, 0)
        dma_in(1, 1)
        body(0, 0, True, True)
        body(1, 1, True, True)

        def mid(t, carry):
            body(2 * t, 0, False, True)
            body(2 * t + 1, 1, False, True)
            return carry

        lax.fori_loop(1, 10, mid, 0)
        body(20, 0, False, True)
        body(21, 1, False, False)
        body(22, 0, False, False)
        wait_out(1)
        wait_out(0)

    return k(emb_table)


def _sc_sum(word_indices, table_bf):
    B, S = word_indices.shape
    E = table_bf.shape[1]
    NC, NS = 2, 16
    NW = NC * NS
    R = B // NW
    GV = E // (2 * _LANES)
    C0 = 104
    C1 = S - C0
    assert C0 % 8 == 0 and C1 <= 128 and S % 8 == 0
    NB = _NBUF
    MAIN = (R // NB) * NB

    mesh = plsc.VectorSubcoreMesh(core_axis_name="c", subcore_axis_name="s")

    @functools.partial(
        pl.kernel,
        out_type=jax.ShapeDtypeStruct((B, E), jnp.float32),
        mesh=mesh,
        compiler_params=pltpu.CompilerParams(
            use_tc_tiling_on_sc=False, needs_layout_passes=False),
        scratch_types=[
            pltpu.VMEM((R, S), jnp.int32),
            pltpu.VMEM((NB, S, E), jnp.bfloat16),
            pltpu.VMEM((R, E), jnp.float32),
        ] + [pltpu.SemaphoreType.DMA] * NB,
    )
    def k(idx_hbm, table_hbm, out_hbm, idx_v, rows_v, sums_v, *sems):
        wid = lax.axis_index("s") * NC + lax.axis_index("c")
        base = wid * R

        pltpu.sync_copy(idx_hbm.at[pl.ds(base, R)], idx_v)

        def gather_row(row, buf):
            sem = sems[buf]
            pltpu.async_copy(
                table_hbm.at[idx_v.at[row, pl.ds(0, C0)]],
                rows_v.at[buf, pl.ds(0, C0), :], sem)
            pltpu.async_copy(
                table_hbm.at[idx_v.at[row, pl.ds(C0, C1)]],
                rows_v.at[buf, pl.ds(C0, C1), :], sem)

        def wait_row(buf):
            sem = sems[buf]
            pltpu.make_async_copy(
                table_hbm.at[idx_v.at[0, pl.ds(0, C0)]],
                rows_v.at[buf, pl.ds(0, C0), :], sem).wait()
            pltpu.make_async_copy(
                table_hbm.at[idx_v.at[0, pl.ds(C0, C1)]],
                rows_v.at[buf, pl.ds(C0, C1), :], sem).wait()

        def accumulate(row, buf):
            def body(t, acc):
                acc = list(acc)
                for j in range(8):
                    s = t * 8 + j
                    for g in range(GV):
                        w = rows_v[buf, s, pl.ds(g * 2 * _LANES, 2 * _LANES)]
                        lo, hi = plsc.unpack(
                            w, format=plsc.PackFormat.INTERLEAVED)
                        acc[2 * g] = acc[2 * g] + lo.astype(jnp.float32)
                        acc[2 * g + 1] = acc[2 * g + 1] + hi.astype(
                            jnp.float32)
                return tuple(acc)
            acc = lax.fori_loop(
                0, S // 8, body,
                tuple(jnp.zeros((_LANES,), jnp.float32)
                      for _ in range(2 * GV)))
            for e in range(2 * GV):
                sums_v[row, pl.ds(e * _LANES, _LANES)] = acc[e]

        for b in range(NB):
            gather_row(b, b)

        def outer(g, carry):
            for b in range(NB):
                row = g + b
                wait_row(b)
                accumulate(row, b)
                gather_row(jnp.minimum(row + NB, R - 1), b)
            return carry

        lax.fori_loop(0, MAIN // NB, lambda t, c: outer(t * NB, c), 0)

        for b in range(NB):
            row = MAIN + b
            wait_row(b)
            if row < R:
                accumulate(row, b)

        pltpu.sync_copy(sums_v, out_hbm.at[pl.ds(base, R)])

    return k(word_indices, table_bf)


def _tc_finish(sums, word_indices, emb0, W1, b1, W2, b2):
    B, S = word_indices.shape
    E = sums.shape[1]
    H = W1.shape[1]
    C = W2.shape[1]
    BB = B
    grid = B // BB

    def body(sums_ref, idx_ref, emb0_ref, w1_ref, b1_ref, w2_ref, b2_ref,
             out_ref):
        idx = idx_ref[...]
        cnt = jnp.sum((idx != 0).astype(jnp.float32), axis=1, keepdims=True)
        npad = float(S) - cnt
        summed = sums_ref[...] - npad * emb0_ref[...]
        avg = jnp.where(cnt > 0, summed / jnp.maximum(cnt, 1.0), 0.0)
        hidden = jnp.maximum(
            jnp.dot(avg, w1_ref[...], preferred_element_type=jnp.float32)
            + b1_ref[...], 0.0)
        out_ref[...] = (
            jnp.dot(hidden, w2_ref[...], preferred_element_type=jnp.float32)
            + b2_ref[...])

    return pl.pallas_call(
        body,
        grid=(grid,),
        in_specs=[
            pl.BlockSpec((BB, E), lambda i: (i, 0)),
            pl.BlockSpec((BB, S), lambda i: (i, 0)),
            pl.BlockSpec((1, E), lambda i: (0, 0)),
            pl.BlockSpec((E, H), lambda i: (0, 0)),
            pl.BlockSpec((1, H), lambda i: (0, 0)),
            pl.BlockSpec((H, C), lambda i: (0, 0)),
            pl.BlockSpec((1, C), lambda i: (0, 0)),
        ],
        out_specs=pl.BlockSpec((BB, C), lambda i: (i, 0)),
        out_shape=jax.ShapeDtypeStruct((B, C), jnp.float32),
    )(sums, word_indices, emb0, W1, b1, W2, b2)


def _inv_perm(E):
    p = [0] * E
    for g in range(E // 32):
        for k in range(16):
            p[32 * g + k] = 32 * g + 2 * k
            p[32 * g + 16 + k] = 32 * g + 2 * k + 1
    return p


def kernel(word_indices, emb_table, W1, b1, W2, b2):
    idx = word_indices.astype(jnp.int32)
    E = emb_table.shape[1]
    table_bf = _sc_pack(emb_table)
    sums = _sc_sum(idx, table_bf)
    invp = jnp.array(_inv_perm(E), dtype=jnp.int32)
    emb0 = table_bf[0:1].astype(jnp.float32)[:, invp]
    return _tc_finish(sums, idx, emb0, W1, b1.reshape(1, -1),
                      W2, b2.reshape(1, -1))


# speedup vs baseline: 2.7760x; 1.0058x over previous
import functools

import jax
import jax.numpy as jnp
from jax import lax
from jax.experimental import pallas as pl
from jax.experimental.pallas import tpu as pltpu
from jax.experimental.pallas import tpu_sc as plsc

_LANES = 16
_NBUF = 2


def _sc_sum(word_indices, emb_table):
    B, S = word_indices.shape
    _, E = emb_table.shape
    NC, NS = 2, 16
    NW = NC * NS
    R = B // NW
    EV = E // _LANES
    C0 = 104
    C1 = S - C0
    assert C0 % 8 == 0 and C1 <= 128 and S % 8 == 0
    NB = _NBUF
    MAIN = (R // NB) * NB

    mesh = plsc.VectorSubcoreMesh(core_axis_name="c", subcore_axis_name="s")

    @functools.partial(
        pl.kernel,
        out_type=jax.ShapeDtypeStruct((B, E), jnp.float32),
        mesh=mesh,
        compiler_params=pltpu.CompilerParams(
            use_tc_tiling_on_sc=False, needs_layout_passes=False),
        scratch_types=[
            pltpu.VMEM((R, S), jnp.int32),
            pltpu.VMEM((NB, S, E), jnp.float32),
            pltpu.VMEM((R, E), jnp.float32),
        ] + [pltpu.SemaphoreType.DMA] * NB,
    )
    def k(idx_hbm, table_hbm, out_hbm, idx_v, rows_v, sums_v, *sems):
        wid = lax.axis_index("s") * NC + lax.axis_index("c")
        base = wid * R

        pltpu.sync_copy(idx_hbm.at[pl.ds(base, R)], idx_v)

        def gather_row(row, buf):
            sem = sems[buf]
            pltpu.async_copy(
                table_hbm.at[idx_v.at[row, pl.ds(0, C0)]],
                rows_v.at[buf, pl.ds(0, C0), :], sem)
            pltpu.async_copy(
                table_hbm.at[idx_v.at[row, pl.ds(C0, C1)]],
                rows_v.at[buf, pl.ds(C0, C1), :], sem)

        def wait_row(buf):
            sem = sems[buf]
            pltpu.make_async_copy(
                table_hbm.at[idx_v.at[0, pl.ds(0, C0)]],
                rows_v.at[buf, pl.ds(0, C0), :], sem).wait()
            pltpu.make_async_copy(
                table_hbm.at[idx_v.at[0, pl.ds(C0, C1)]],
                rows_v.at[buf, pl.ds(C0, C1), :], sem).wait()

        def accumulate(row, buf):
            def body(t, acc):
                acc = list(acc)
                for j in range(8):
                    s = t * 8 + j
                    for e in range(EV):
                        acc[e] = acc[e] + rows_v[
                            buf, s, pl.ds(e * _LANES, _LANES)]
                return tuple(acc)
            acc = lax.fori_loop(
                0, S // 8, body,
                tuple(jnp.zeros((_LANES,), jnp.float32) for _ in range(EV)))
            for e in range(EV):
                sums_v[row, pl.ds(e * _LANES, _LANES)] = acc[e]

        for b in range(NB):
            gather_row(b, b)

        def outer(g, carry):
            for b in range(NB):
                row = g + b
                wait_row(b)
                accumulate(row, b)
                gather_row(jnp.minimum(row + NB, R - 1), b)
            return carry

        lax.fori_loop(0, MAIN // NB, lambda t, c: outer(t * NB, c), 0)

        for b in range(NB):
            row = MAIN + b
            wait_row(b)
            if row < R:
                accumulate(row, b)

        pltpu.sync_copy(sums_v, out_hbm.at[pl.ds(base, R)])

    return k(word_indices, emb_table)


def _tc_finish(sums, word_indices, emb0, W1, b1, W2, b2):
    B, S = word_indices.shape
    E = sums.shape[1]
    H = W1.shape[1]
    C = W2.shape[1]
    BB = B
    grid = B // BB

    def body(sums_ref, idx_ref, emb0_ref, w1_ref, b1_ref, w2_ref, b2_ref,
             out_ref):
        idx = idx_ref[...]
        cnt = jnp.sum((idx != 0).astype(jnp.float32), axis=1, keepdims=True)
        npad = float(S) - cnt
        summed = sums_ref[...] - npad * emb0_ref[...]
        avg = jnp.where(cnt > 0, summed / jnp.maximum(cnt, 1.0), 0.0)
        hidden = jnp.maximum(
            jnp.dot(avg, w1_ref[...], preferred_element_type=jnp.float32)
            + b1_ref[...], 0.0)
        out_ref[...] = (
            jnp.dot(hidden, w2_ref[...], preferred_element_type=jnp.float32)
            + b2_ref[...])

    return pl.pallas_call(
        body,
        grid=(grid,),
        in_specs=[
            pl.BlockSpec((BB, E), lambda i: (i, 0)),
            pl.BlockSpec((BB, S), lambda i: (i, 0)),
            pl.BlockSpec((1, E), lambda i: (0, 0)),
            pl.BlockSpec((E, H), lambda i: (0, 0)),
            pl.BlockSpec((1, H), lambda i: (0, 0)),
            pl.BlockSpec((H, C), lambda i: (0, 0)),
            pl.BlockSpec((1, C), lambda i: (0, 0)),
        ],
        out_specs=pl.BlockSpec((BB, C), lambda i: (i, 0)),
        out_shape=jax.ShapeDtypeStruct((B, C), jnp.float32),
    )(sums, word_indices, emb0, W1, b1, W2, b2)


def kernel(word_indices, emb_table, W1, b1, W2, b2):
    idx = word_indices.astype(jnp.int32)
    sums = _sc_sum(idx, emb_table)
    return _tc_finish(sums, idx, emb_table[0:1], W1, b1.reshape(1, -1),
                      W2, b2.reshape(1, -1))


# speedup vs baseline: 3.3817x; 1.2182x over previous
import functools

import jax
import jax.numpy as jnp
from jax import lax
from jax.experimental import pallas as pl
from jax.experimental.pallas import tpu as pltpu
from jax.experimental.pallas import tpu_sc as plsc

_LANES = 16
_NBUF = 3


def _sc_sum(word_indices, emb_table):
    B, S = word_indices.shape
    _, E = emb_table.shape
    NC, NS = 2, 16
    NW = NC * NS
    R = B // NW
    EV = E // _LANES
    C0 = 104
    C1 = S - C0
    assert C0 % 8 == 0 and C1 <= 128 and S % 8 == 0
    NB = _NBUF
    MAIN = (R // NB) * NB

    mesh = plsc.VectorSubcoreMesh(core_axis_name="c", subcore_axis_name="s")

    @functools.partial(
        pl.kernel,
        out_type=jax.ShapeDtypeStruct((B, E), jnp.float32),
        mesh=mesh,
        compiler_params=pltpu.CompilerParams(
            use_tc_tiling_on_sc=False, needs_layout_passes=False),
        scratch_types=[
            pltpu.VMEM((R, S), jnp.int32),
            pltpu.VMEM((NB, S, E), jnp.float32),
            pltpu.VMEM((R, E), jnp.float32),
        ] + [pltpu.SemaphoreType.DMA] * NB,
    )
    def k(idx_hbm, table_hbm, out_hbm, idx_v, rows_v, sums_v, *sems):
        wid = lax.axis_index("s") * NC + lax.axis_index("c")
        base = wid * R

        pltpu.sync_copy(idx_hbm.at[pl.ds(base, R)], idx_v)

        def gather_row(row, buf):
            sem = sems[buf]
            pltpu.async_copy(
                table_hbm.at[idx_v.at[row, pl.ds(0, C0)]],
                rows_v.at[buf, pl.ds(0, C0), :], sem)
            pltpu.async_copy(
                table_hbm.at[idx_v.at[row, pl.ds(C0, C1)]],
                rows_v.at[buf, pl.ds(C0, C1), :], sem)

        def wait_row(buf):
            sem = sems[buf]
            pltpu.make_async_copy(
                table_hbm.at[idx_v.at[0, pl.ds(0, C0)]],
                rows_v.at[buf, pl.ds(0, C0), :], sem).wait()
            pltpu.make_async_copy(
                table_hbm.at[idx_v.at[0, pl.ds(C0, C1)]],
                rows_v.at[buf, pl.ds(C0, C1), :], sem).wait()

        def accumulate(row, buf):
            def body(t, acc):
                acc = list(acc)
                for j in range(8):
                    s = t * 8 + j
                    for e in range(EV):
                        acc[e] = acc[e] + rows_v[
                            buf, s, pl.ds(e * _LANES, _LANES)]
                return tuple(acc)
            acc = lax.fori_loop(
                0, S // 8, body,
                tuple(jnp.zeros((_LANES,), jnp.float32) for _ in range(EV)))
            for e in range(EV):
                sums_v[row, pl.ds(e * _LANES, _LANES)] = acc[e]

        for b in range(NB):
            gather_row(b, b)

        def outer(g, carry):
            for b in range(NB):
                row = g + b
                wait_row(b)
                accumulate(row, b)
                gather_row(jnp.minimum(row + NB, R - 1), b)
            return carry

        lax.fori_loop(0, MAIN // NB, lambda t, c: outer(t * NB, c), 0)

        for b in range(NB):
            row = MAIN + b
            wait_row(b)
            if row < R:
                accumulate(row, b)

        pltpu.sync_copy(sums_v, out_hbm.at[pl.ds(base, R)])

    return k(word_indices, emb_table)


def _tc_finish(sums, word_indices, emb0, W1, b1, W2, b2):
    B, S = word_indices.shape
    E = sums.shape[1]
    H = W1.shape[1]
    C = W2.shape[1]
    BB = B
    grid = B // BB

    def body(sums_ref, idx_ref, emb0_ref, w1_ref, b1_ref, w2_ref, b2_ref,
             out_ref):
        idx = idx_ref[...]
        cnt = jnp.sum((idx != 0).astype(jnp.float32), axis=1, keepdims=True)
        npad = float(S) - cnt
        summed = sums_ref[...] - npad * emb0_ref[...]
        avg = jnp.where(cnt > 0, summed / jnp.maximum(cnt, 1.0), 0.0)
        hidden = jnp.maximum(
            jnp.dot(avg, w1_ref[...], preferred_element_type=jnp.float32)
            + b1_ref[...], 0.0)
        out_ref[...] = (
            jnp.dot(hidden, w2_ref[...], preferred_element_type=jnp.float32)
            + b2_ref[...])

    return pl.pallas_call(
        body,
        grid=(grid,),
        in_specs=[
            pl.BlockSpec((BB, E), lambda i: (i, 0)),
            pl.BlockSpec((BB, S), lambda i: (i, 0)),
            pl.BlockSpec((1, E), lambda i: (0, 0)),
            pl.BlockSpec((E, H), lambda i: (0, 0)),
            pl.BlockSpec((1, H), lambda i: (0, 0)),
            pl.BlockSpec((H, C), lambda i: (0, 0)),
            pl.BlockSpec((1, C), lambda i: (0, 0)),
        ],
        out_specs=pl.BlockSpec((BB, C), lambda i: (i, 0)),
        out_shape=jax.ShapeDtypeStruct((B, C), jnp.float32),
    )(sums, word_indices, emb0, W1, b1, W2, b2)


def kernel(word_indices, emb_table, W1, b1, W2, b2):
    idx = word_indices.astype(jnp.int32)
    sums = _sc_sum(idx, emb_table)
    return _tc_finish(sums, idx, emb_table[0:1], W1, b1.reshape(1, -1),
                      W2, b2.reshape(1, -1))
